# baseline (speedup 1.0000x reference)
import functools

import jax
import jax.numpy as jnp
from jax import lax
from jax.experimental import pallas as pl
from jax.experimental.pallas import tpu as pltpu
from jax.experimental.pallas import tpu_sc as plsc

N = 10000
E = 320000
NC = 2
NS = 16
NW = NC * NS
EPW = E // NW
CH = 112
NCHUNK = 96
EPAD = NW * NCHUNK * CH
RPAD = 10240
RPS = RPAD // NS
NPAD = 10240
DPS = NPAD // NS

_mesh = plsc.VectorSubcoreMesh(core_axis_name="c", subcore_axis_name="s")


@functools.partial(
    pl.kernel,
    out_type=jax.ShapeDtypeStruct((NC, NPAD), jnp.float32),
    mesh=_mesh,
    scratch_types=[
        pltpu.VMEM((NCHUNK, CH), jnp.int32),
        pltpu.VMEM((NCHUNK, CH), jnp.float32),
        pltpu.VMEM((DPS,), jnp.float32),
        pltpu.VMEM_SHARED((NPAD,), jnp.float32),
    ],
)
def _sc_degree(col_h, w_h, out_h, colv, wv, zb, acc):
    c = lax.axis_index("c")
    s = lax.axis_index("s")
    wid = c * NS + s

    @pl.loop(0, DPS // 16)
    def _zero(i):
        zb[pl.ds(i * 16, 16)] = jnp.zeros((16,), jnp.float32)

    pltpu.sync_copy(zb, acc.at[pl.ds(s * DPS, DPS)])
    plsc.subcore_barrier()

    pltpu.sync_copy(col_h.at[wid], colv)
    pltpu.sync_copy(w_h.at[wid], wv)

    @pl.loop(0, NCHUNK)
    def _scatter(j):
        pltpu.sync_copy(wv.at[j], acc.at[colv.at[j]], add=True)

    plsc.subcore_barrier()
    pltpu.sync_copy(acc.at[pl.ds(s * DPS, DPS)], zb)
    pltpu.sync_copy(zb, out_h.at[c, pl.ds(s * DPS, DPS)])


NB = 8
GA = 4

@functools.partial(
    pl.kernel,
    out_type=jax.ShapeDtypeStruct((NC, RPAD, 64), jnp.float32),
    mesh=_mesh,
    compiler_params=pltpu.CompilerParams(use_tc_tiling_on_sc=False),
    scratch_types=[
        pltpu.VMEM((NCHUNK, CH), jnp.int32),
        pltpu.VMEM((NCHUNK, CH), jnp.int32),
        pltpu.VMEM((NCHUNK, CH), jnp.float32),
    ]
    + [pltpu.VMEM((CH, 64), jnp.float32)] * NB
    + [pltpu.VMEM_SHARED((RPAD, 64), jnp.float32)]
    + [pltpu.SemaphoreType.DMA] * 2,
)
def _sc_edges(hs_h, row_h, col_h, w_h, out_h, rowv, colv, wv, *rest):
    rb = rest[:NB]
    acc = rest[NB]
    gsem = rest[NB + 1]
    ssem = rest[NB + 2]
    c = lax.axis_index("c")
    s = lax.axis_index("s")
    wid = c * NS + s

    @pl.loop(0, 80)
    def _zero(r):
        for q in range(4):
            rb[0][r, pl.ds(q * 16, 16)] = jnp.zeros((16,), jnp.float32)

    for p in range(RPS // 80):
        pltpu.sync_copy(rb[0].at[pl.ds(0, 80)], acc.at[pl.ds(s * RPS + p * 80, 80)])
    plsc.subcore_barrier()

    pltpu.sync_copy(row_h.at[wid], rowv)
    pltpu.sync_copy(col_h.at[wid], colv)
    pltpu.sync_copy(w_h.at[wid], wv)

    def _mul(b, j):
        @pl.loop(0, CH // 16)
        def _edge16(g):
            w16 = wv[j, pl.ds(g * 16, 16)]
            for t in range(16):
                e = g * 16 + t
                wspl = lax.gather(
                    w16,
                    jnp.full((16, 1), t, jnp.int32),
                    lax.GatherDimensionNumbers(
                        offset_dims=(),
                        collapsed_slice_dims=(0,),
                        start_index_map=(0,),
                    ),
                    slice_sizes=(1,),
                    mode=lax.GatherScatterMode.PROMISE_IN_BOUNDS,
                )
                for q in range(4):
                    sl = pl.ds(q * 16, 16)
                    rb[b][e, sl] = rb[b][e, sl] * wspl

    def _wait(sem, b):
        pltpu.make_async_copy(hs_h.at[pl.ds(0, CH)], rb[b], sem).wait()

    for j in range(GA):
        pltpu.async_copy(hs_h.at[rowv.at[j]], rb[j], gsem)

    @pl.loop(0, NCHUNK // NB)
    def _chunk(k):
        for i in range(NB):
            j = NB * k + i
            _wait(gsem, i)
            _mul(i, j)
            pltpu.async_copy(rb[i], acc.at[colv.at[j]], ssem, add=True)
            bn = (i + GA) % NB
            jn = j + GA

            @pl.when((j >= NB - GA) & (jn < NCHUNK))
            def _():
                _wait(ssem, bn)

            @pl.when(jn < NCHUNK)
            def _():
                pltpu.async_copy(hs_h.at[rowv.at[jn]], rb[bn], gsem)

    for i in range(NB):
        _wait(ssem, i)

    plsc.subcore_barrier()
    for p in range(RPS // 80):
        pltpu.sync_copy(acc.at[pl.ds(s * RPS + p * 80, 80)], rb[0].at[pl.ds(0, 80)])
        pltpu.sync_copy(rb[0].at[pl.ds(0, 80)], out_h.at[c, pl.ds(s * RPS + p * 80, 80)])


_BR = 2000
_G = N // _BR


def _tc0_body(d0, d1, x, w1, dis, hs1):
    deg = d0[...] + d1[...]
    div = jnp.where(deg > 0, lax.rsqrt(jnp.where(deg > 0, deg, 1.0)), 0.0)
    dis[...] = div
    h = jnp.dot(x[...], w1[...], preferred_element_type=jnp.float32)
    hs1[...] = h * div


def _tc0(d0, d1, x, w1):
    return pl.pallas_call(
        _tc0_body,
        grid=(_G,),
        in_specs=[
            pl.BlockSpec((_BR, 1), lambda i: (i, 0)),
            pl.BlockSpec((_BR, 1), lambda i: (i, 0)),
            pl.BlockSpec((_BR, 128), lambda i: (i, 0)),
            pl.BlockSpec((128, 64), lambda i: (0, 0)),
        ],
        out_specs=[
            pl.BlockSpec((_BR, 1), lambda i: (i, 0)),
            pl.BlockSpec((_BR, 64), lambda i: (i, 0)),
        ],
        out_shape=[
            jax.ShapeDtypeStruct((N, 1), jnp.float32),
            jax.ShapeDtypeStruct((N, 64), jnp.float32),
        ],
    )(d0, d1, x, w1)


def _tc_mid_body(p0, p1, dis, b, wn, hk, hsn):
    div = dis[...]
    hkv = jnp.maximum(div * (p0[...] + p1[...]) + b[...], 0.0)
    hk[...] = hkv
    hsn[...] = jnp.dot(hkv, wn[...], preferred_element_type=jnp.float32) * div


def _tc_mid(p0, p1, dis, b, wn):
    return pl.pallas_call(
        _tc_mid_body,
        grid=(_G,),
        in_specs=[
            pl.BlockSpec((_BR, 64), lambda i: (i, 0)),
            pl.BlockSpec((_BR, 64), lambda i: (i, 0)),
            pl.BlockSpec((_BR, 1), lambda i: (i, 0)),
            pl.BlockSpec((1, 64), lambda i: (0, 0)),
            pl.BlockSpec((64, 64), lambda i: (0, 0)),
        ],
        out_specs=[
            pl.BlockSpec((_BR, 64), lambda i: (i, 0)),
            pl.BlockSpec((_BR, 64), lambda i: (i, 0)),
        ],
        out_shape=[
            jax.ShapeDtypeStruct((N, 64), jnp.float32),
            jax.ShapeDtypeStruct((N, 64), jnp.float32),
        ],
    )(p0, p1, dis, b, wn)


def _tc_last_body(p0, p1, dis, b, h3):
    h3[...] = dis[...] * (p0[...] + p1[...]) + b[...]


def _tc_last(p0, p1, dis, b):
    return pl.pallas_call(
        _tc_last_body,
        grid=(_G,),
        in_specs=[
            pl.BlockSpec((_BR, 64), lambda i: (i, 0)),
            pl.BlockSpec((_BR, 64), lambda i: (i, 0)),
            pl.BlockSpec((_BR, 1), lambda i: (i, 0)),
            pl.BlockSpec((1, 64), lambda i: (0, 0)),
        ],
        out_specs=pl.BlockSpec((_BR, 64), lambda i: (i, 0)),
        out_shape=jax.ShapeDtypeStruct((N, 64), jnp.float32),
    )(p0, p1, dis, b)


@jax.jit
def kernel(x, edge_index, edge_weight, W1, b1, W2, b2, W3, b3):
    pad = EPAD - E
    row = jnp.pad(edge_index[0], (0, pad)).reshape(NW, NCHUNK, CH)
    col = jnp.pad(edge_index[1], (0, pad)).reshape(NW, NCHUNK, CH)
    wr = jnp.pad(edge_weight, (0, pad)).reshape(NW, NCHUNK, CH)

    dp = _sc_degree(col, wr)
    d0 = dp[0, :N].reshape(N, 1)
    d1 = dp[1, :N].reshape(N, 1)

    dis, hs1 = _tc0(d0, d1, x, W1)
    p = _sc_edges(hs1, row, col, wr)
    h1, hs2 = _tc_mid(p[0, :N], p[1, :N], dis, b1.reshape(1, 64), W2)
    p = _sc_edges(hs2, row, col, wr)
    h2, hs3 = _tc_mid(p[0, :N], p[1, :N], dis, b2.reshape(1, 64), W3)
    p = _sc_edges(hs3, row, col, wr)
    h3 = _tc_last(p[0, :N], p[1, :N], dis, b3.reshape(1, 64))
    return jnp.stack([h1, h2, h3], axis=0)


# speedup vs baseline: 3.0147x; 3.0147x over previous
import functools

import jax
import jax.numpy as jnp
from jax import lax
from jax.experimental import pallas as pl
from jax.experimental.pallas import tpu as pltpu
from jax.experimental.pallas import tpu_sc as plsc

N = 10000
E = 320000
NC = 2
NS = 16
NW = NC * NS
EPW = E // NW
CH = 128
NCHUNK = 80
EPAD = NW * NCHUNK * CH
RPAD = 10240
RPS = RPAD // NS
NPAD = 10240
DPS = NPAD // NS

_mesh = plsc.VectorSubcoreMesh(core_axis_name="c", subcore_axis_name="s")


@functools.partial(
    pl.kernel,
    out_type=jax.ShapeDtypeStruct((NC, NPAD), jnp.float32),
    mesh=_mesh,
    scratch_types=[
        pltpu.VMEM((NCHUNK, CH), jnp.int32),
        pltpu.VMEM((NCHUNK, CH), jnp.float32),
        pltpu.VMEM((DPS,), jnp.float32),
        pltpu.VMEM_SHARED((NPAD,), jnp.float32),
    ],
)
def _sc_degree(col_h, w_h, out_h, colv, wv, zb, acc):
    c = lax.axis_index("c")
    s = lax.axis_index("s")
    wid = c * NS + s

    @pl.loop(0, DPS // 16)
    def _zero(i):
        zb[pl.ds(i * 16, 16)] = jnp.zeros((16,), jnp.float32)

    pltpu.sync_copy(zb, acc.at[pl.ds(s * DPS, DPS)])
    plsc.subcore_barrier()

    pltpu.sync_copy(col_h.at[wid], colv)
    pltpu.sync_copy(w_h.at[wid], wv)

    @pl.loop(0, NCHUNK)
    def _scatter(j):
        pltpu.sync_copy(wv.at[j], acc.at[colv.at[j]], add=True)

    plsc.subcore_barrier()
    pltpu.sync_copy(acc.at[pl.ds(s * DPS, DPS)], zb)
    pltpu.sync_copy(zb, out_h.at[c, pl.ds(s * DPS, DPS)])


@functools.partial(
    pl.kernel,
    out_type=jax.ShapeDtypeStruct((NC, RPAD, 64), jnp.float32),
    mesh=_mesh,
    compiler_params=pltpu.CompilerParams(use_tc_tiling_on_sc=False, needs_layout_passes=False),
    scratch_types=[
        pltpu.VMEM((NCHUNK, CH), jnp.int32),
        pltpu.VMEM((NCHUNK, CH), jnp.int32),
        pltpu.VMEM((NCHUNK, CH), jnp.float32),
        pltpu.VMEM((CH, 64), jnp.bfloat16),
        pltpu.VMEM((CH, 64), jnp.bfloat16),
        pltpu.VMEM((CH, 64), jnp.float32),
        pltpu.VMEM((CH, 64), jnp.float32),
        pltpu.VMEM_SHARED((RPAD, 64), jnp.float32),
        pltpu.SemaphoreType.DMA,
        pltpu.SemaphoreType.DMA,
        pltpu.SemaphoreType.DMA,
        pltpu.SemaphoreType.DMA,
    ],
)
def _sc_edges(hs_h, row_h, col_h, w_h, out_h, rowv, colv, wv, gb0, gb1, rb0, rb1,
              acc, gs0, gs1, ss0, ss1):
    c = lax.axis_index("c")
    s = lax.axis_index("s")
    wid = c * NS + s

    @pl.loop(0, CH)
    def _zero(r):
        for g in range(4):
            rb0[r, pl.ds(g * 16, 16)] = jnp.zeros((16,), jnp.float32)

    for p in range(RPS // CH):
        pltpu.sync_copy(rb0, acc.at[pl.ds(s * RPS + p * CH, CH)])
    plsc.subcore_barrier()

    pltpu.sync_copy(row_h.at[wid], rowv)
    pltpu.sync_copy(col_h.at[wid], colv)
    pltpu.sync_copy(w_h.at[wid], wv)

    def _mul(gb, rb, j):
        @pl.loop(0, CH // 16)
        def _edge16(g):
            w16 = wv[j, pl.ds(g * 16, 16)]
            for t in range(16):
                e = g * 16 + t
                wspl = lax.gather(
                    w16,
                    jnp.full((16, 1), t, jnp.int32),
                    lax.GatherDimensionNumbers(
                        offset_dims=(),
                        collapsed_slice_dims=(0,),
                        start_index_map=(0,),
                    ),
                    slice_sizes=(1,),
                    mode=lax.GatherScatterMode.PROMISE_IN_BOUNDS,
                )
                for q in range(2):
                    v32 = gb[e, pl.ds(q * 32, 32)]
                    lo, hi = plsc.unpack(v32, format=plsc.PackFormat.INTERLEAVED)
                    rb[e, pl.ds(q * 32, 16)] = lo * wspl
                    rb[e, pl.ds(q * 32 + 16, 16)] = hi * wspl

    def _wait(sem, rb):
        pltpu.make_async_copy(hs_h.at[pl.ds(0, CH)], gb0, sem).wait()

    def _swait(sem):
        pltpu.make_async_copy(acc.at[pl.ds(0, CH)], rb0, sem).wait()

    pltpu.async_copy(hs_h.at[rowv.at[0]], gb0, gs0)

    @pl.loop(0, NCHUNK // 2)
    def _chunk(k):
        j0 = 2 * k
        pltpu.async_copy(hs_h.at[rowv.at[j0 + 1]], gb1, gs1)
        _wait(gs0, gb0)
        @pl.when(k > 0)
        def _():
            _swait(ss0)
        _mul(gb0, rb0, j0)
        pltpu.async_copy(rb0, acc.at[colv.at[j0]], ss0, add=True)
        @pl.when(k < NCHUNK // 2 - 1)
        def _():
            pltpu.async_copy(hs_h.at[rowv.at[j0 + 2]], gb0, gs0)
        _wait(gs1, gb1)
        @pl.when(k > 0)
        def _():
            _swait(ss1)
        _mul(gb1, rb1, j0 + 1)
        pltpu.async_copy(rb1, acc.at[colv.at[j0 + 1]], ss1, add=True)

    _swait(ss0)
    _swait(ss1)

    plsc.subcore_barrier()
    for p in range(RPS // CH):
        pltpu.sync_copy(acc.at[pl.ds(s * RPS + p * CH, CH)], rb0)
        pltpu.sync_copy(rb0, out_h.at[c, pl.ds(s * RPS + p * CH, CH)])


_BR = 2000
_G = N // _BR


def _tc0_body(d0, d1, x, w1, dis, hs1):
    deg = d0[...] + d1[...]
    div = jnp.where(deg > 0, lax.rsqrt(jnp.where(deg > 0, deg, 1.0)), 0.0)
    dis[...] = div
    h = jnp.dot(x[...], w1[...], preferred_element_type=jnp.float32)
    hs1[...] = (h * div).astype(jnp.bfloat16)


def _tc0(d0, d1, x, w1):
    return pl.pallas_call(
        _tc0_body,
        grid=(_G,),
        in_specs=[
            pl.BlockSpec((_BR, 1), lambda i: (i, 0)),
            pl.BlockSpec((_BR, 1), lambda i: (i, 0)),
            pl.BlockSpec((_BR, 128), lambda i: (i, 0)),
            pl.BlockSpec((128, 64), lambda i: (0, 0)),
        ],
        out_specs=[
            pl.BlockSpec((_BR, 1), lambda i: (i, 0)),
            pl.BlockSpec((_BR, 64), lambda i: (i, 0)),
        ],
        out_shape=[
            jax.ShapeDtypeStruct((N, 1), jnp.float32),
            jax.ShapeDtypeStruct((N, 64), jnp.bfloat16),
        ],
    )(d0, d1, x, w1)


def _tc_mid_body(p0, p1, dis, b, wn, hk, hsn):
    div = dis[...]
    hkv = jnp.maximum(div * (p0[...] + p1[...]) + b[...], 0.0)
    hk[...] = hkv
    hsn[...] = (jnp.dot(hkv, wn[...], preferred_element_type=jnp.float32) * div
                ).astype(jnp.bfloat16)


def _tc_mid(p0, p1, dis, b, wn):
    return pl.pallas_call(
        _tc_mid_body,
        grid=(_G,),
        in_specs=[
            pl.BlockSpec((_BR, 64), lambda i: (i, 0)),
            pl.BlockSpec((_BR, 64), lambda i: (i, 0)),
            pl.BlockSpec((_BR, 1), lambda i: (i, 0)),
            pl.BlockSpec((1, 64), lambda i: (0, 0)),
            pl.BlockSpec((64, 64), lambda i: (0, 0)),
        ],
        out_specs=[
            pl.BlockSpec((_BR, 64), lambda i: (i, 0)),
            pl.BlockSpec((_BR, 64), lambda i: (i, 0)),
        ],
        out_shape=[
            jax.ShapeDtypeStruct((N, 64), jnp.float32),
            jax.ShapeDtypeStruct((N, 64), jnp.bfloat16),
        ],
    )(p0, p1, dis, b, wn)


def _tc_last_body(p0, p1, dis, b, h3):
    h3[...] = dis[...] * (p0[...] + p1[...]) + b[...]


def _tc_last(p0, p1, dis, b):
    return pl.pallas_call(
        _tc_last_body,
        grid=(_G,),
        in_specs=[
            pl.BlockSpec((_BR, 64), lambda i: (i, 0)),
            pl.BlockSpec((_BR, 64), lambda i: (i, 0)),
            pl.BlockSpec((_BR, 1), lambda i: (i, 0)),
            pl.BlockSpec((1, 64), lambda i: (0, 0)),
        ],
        out_specs=pl.BlockSpec((_BR, 64), lambda i: (i, 0)),
        out_shape=jax.ShapeDtypeStruct((N, 64), jnp.float32),
    )(p0, p1, dis, b)


_PERM = jnp.array(
    [32 * q + 16 * r + i for q in range(2) for i in range(16) for r in range(2)],
    dtype=jnp.int32,
)


@jax.jit
def kernel(x, edge_index, edge_weight, W1, b1, W2, b2, W3, b3):
    pad = EPAD - E
    row = jnp.pad(edge_index[0], (0, pad)).reshape(NW, NCHUNK, CH)
    col = jnp.pad(edge_index[1], (0, pad)).reshape(NW, NCHUNK, CH)
    wr = jnp.pad(edge_weight, (0, pad)).reshape(NW, NCHUNK, CH)

    dp = _sc_degree(col, wr)
    d0 = dp[0, :N].reshape(N, 1)
    d1 = dp[1, :N].reshape(N, 1)

    dis, hs1 = _tc0(d0, d1, x, W1)
    p = _sc_edges(hs1[:, _PERM], row, col, wr)
    h1, hs2 = _tc_mid(p[0, :N], p[1, :N], dis, b1.reshape(1, 64), W2)
    p = _sc_edges(hs2[:, _PERM], row, col, wr)
    h2, hs3 = _tc_mid(p[0, :N], p[1, :N], dis, b2.reshape(1, 64), W3)
    p = _sc_edges(hs3[:, _PERM], row, col, wr)
    h3 = _tc_last(p[0, :N], p[1, :N], dis, b3.reshape(1, 64))
    return jnp.stack([h1, h2, h3], axis=0)
